# baseline (speedup 1.0000x reference)
import functools

import jax
import jax.numpy as jnp
from jax import lax
from jax.experimental import pallas as pl
from jax.experimental.pallas import tpu as pltpu
from jax.experimental.pallas import tpu_sc as plsc

E = 500000
N = 100000
D = 128
NC, NS, L = 2, 16, 16
NW = NC * NS
CHUNK = 15744
TAIL = E - (NW - 1) * CHUNK
PAD_IDX = N
H = 102400
HS = H // NS
C = 6400
G = H // C


_sc_mesh = plsc.VectorSubcoreMesh(
    core_axis_name="c", subcore_axis_name="s", num_cores=NC, num_subcores=NS)


@functools.partial(
    pl.kernel,
    out_type=jax.ShapeDtypeStruct((NC, H), jnp.float32),
    mesh=_sc_mesh,
    scratch_types=[
        pltpu.VMEM((CHUNK,), jnp.int32),
        pltpu.VMEM((CHUNK,), jnp.float32),
        pltpu.VMEM((HS,), jnp.float32),
        pltpu.VMEM_SHARED((H,), jnp.float32),
        pltpu.SemaphoreType.DMA,
    ],
    compiler_params=pltpu.CompilerParams(needs_layout_passes=False),
)
def _sc_hist(neighs_hbm, out_hbm, idx_v, ones_v, zbuf_v, hist_sh, sem):
    cid = lax.axis_index("c")
    sid = lax.axis_index("s")
    wid = sid * NC + cid

    @pl.when(wid < NW - 1)
    def _load_full():
        pltpu.async_copy(neighs_hbm.at[pl.ds(wid * CHUNK, CHUNK)], idx_v, sem)

    @pl.when(wid == NW - 1)
    def _load_tail():
        pltpu.async_copy(neighs_hbm.at[pl.ds((NW - 1) * CHUNK, TAIL)],
                         idx_v.at[pl.ds(0, TAIL)], sem)

    ones = jnp.full((L,), 1.0, jnp.float32)
    zeros = jnp.zeros((L,), jnp.float32)

    def fill_body(j, _):
        ones_v[pl.ds(j * L, L)] = ones
        return 0

    lax.fori_loop(0, CHUNK // L, fill_body, 0, unroll=8)

    def zero_body(i, _):
        zbuf_v[pl.ds(i * L, L)] = zeros
        return 0

    lax.fori_loop(0, HS // L, zero_body, 0, unroll=8)
    pltpu.sync_copy(zbuf_v, hist_sh.at[pl.ds(sid * HS, HS)])

    @pl.when(wid < NW - 1)
    def _wait_full():
        pltpu.make_async_copy(
            neighs_hbm.at[pl.ds(wid * CHUNK, CHUNK)], idx_v, sem).wait()

    @pl.when(wid == NW - 1)
    def _wait_tail():
        pltpu.make_async_copy(
            neighs_hbm.at[pl.ds((NW - 1) * CHUNK, TAIL)],
            idx_v.at[pl.ds(0, TAIL)], sem).wait()
        lane = lax.iota(jnp.int32, L)

        def pad_body(i, _):
            off = (i * L) % (H - N - L)
            idx_v[pl.ds(TAIL + i * L, L)] = PAD_IDX + off + lane
            return 0

        lax.fori_loop(0, (CHUNK - TAIL) // L, pad_body, 0, unroll=8)

    plsc.subcore_barrier()

    pltpu.sync_copy(ones_v, hist_sh.at[idx_v], add=True)
    plsc.subcore_barrier()

    @pl.when(sid == 0)
    def _writeback():
        pltpu.sync_copy(hist_sh, out_hbm.at[cid])


def _tc_body(hist_ref, feat_ref, out_ref):
    g = pl.program_id(0)

    @pl.when(g == 0)
    def _init():
        out_ref[...] = jnp.zeros_like(out_ref)

    c = jnp.sum(hist_ref[...], axis=0, keepdims=True)
    f = feat_ref[...]
    row = g * C + lax.broadcasted_iota(jnp.int32, (C, D), 0)
    f = jnp.where(row < N, f, 0.0)
    out_ref[...] += lax.dot_general(
        c, f, (((1,), (0,)), ((), ())), preferred_element_type=jnp.float32)

    @pl.when(g == G - 1)
    def _scale():
        out_ref[...] *= jnp.float32(1.0 / E)


_tc_matvec = pl.pallas_call(
    _tc_body,
    grid=(G,),
    in_specs=[
        pl.BlockSpec((NC, C), lambda g: (0, g)),
        pl.BlockSpec((C, D), lambda g: (g, 0)),
    ],
    out_specs=pl.BlockSpec((1, D), lambda g: (0, 0)),
    out_shape=jax.ShapeDtypeStruct((1, D), jnp.float32),
)


def kernel(u, neighs, features):
    hist = _sc_hist(neighs.astype(jnp.int32))
    out = _tc_matvec(hist, features)
    return out[0]


# speedup vs baseline: 1.0858x; 1.0858x over previous
import functools

import jax
import jax.numpy as jnp
from jax import lax
from jax.experimental import pallas as pl
from jax.experimental.pallas import tpu as pltpu
from jax.experimental.pallas import tpu_sc as plsc

E = 500000
N = 100000
D = 128
NC, NS, L = 2, 16, 16
NW = NC * NS
CHUNK = 15744
TAIL = E - (NW - 1) * CHUNK
PAD_IDX = N
H = 102400
HS = H // NS
C = 12800
G = H // C


_sc_mesh = plsc.VectorSubcoreMesh(
    core_axis_name="c", subcore_axis_name="s", num_cores=NC, num_subcores=NS)


@functools.partial(
    pl.kernel,
    out_type=jax.ShapeDtypeStruct((NC, H), jnp.float32),
    mesh=_sc_mesh,
    scratch_types=[
        pltpu.VMEM((CHUNK,), jnp.int32),
        pltpu.VMEM((CHUNK,), jnp.float32),
        pltpu.VMEM((HS,), jnp.float32),
        pltpu.VMEM_SHARED((H,), jnp.float32),
        pltpu.SemaphoreType.DMA,
    ],
    compiler_params=pltpu.CompilerParams(needs_layout_passes=False),
)
def _sc_hist(neighs_hbm, out_hbm, idx_v, ones_v, zbuf_v, hist_sh, sem):
    cid = lax.axis_index("c")
    sid = lax.axis_index("s")
    wid = sid * NC + cid

    @pl.when(wid < NW - 1)
    def _load_full():
        pltpu.async_copy(neighs_hbm.at[pl.ds(wid * CHUNK, CHUNK)], idx_v, sem)

    @pl.when(wid == NW - 1)
    def _load_tail():
        pltpu.async_copy(neighs_hbm.at[pl.ds((NW - 1) * CHUNK, TAIL)],
                         idx_v.at[pl.ds(0, TAIL)], sem)

    ones = jnp.full((L,), 1.0, jnp.float32)
    zeros = jnp.zeros((L,), jnp.float32)

    def fill_body(j, _):
        ones_v[pl.ds(j * L, L)] = ones
        return 0

    lax.fori_loop(0, CHUNK // L, fill_body, 0, unroll=8)

    def zero_body(i, _):
        zbuf_v[pl.ds(i * L, L)] = zeros
        return 0

    lax.fori_loop(0, HS // L, zero_body, 0, unroll=8)
    pltpu.sync_copy(zbuf_v, hist_sh.at[pl.ds(sid * HS, HS)])

    @pl.when(wid < NW - 1)
    def _wait_full():
        pltpu.make_async_copy(
            neighs_hbm.at[pl.ds(wid * CHUNK, CHUNK)], idx_v, sem).wait()

    @pl.when(wid == NW - 1)
    def _wait_tail():
        pltpu.make_async_copy(
            neighs_hbm.at[pl.ds((NW - 1) * CHUNK, TAIL)],
            idx_v.at[pl.ds(0, TAIL)], sem).wait()
        lane = lax.iota(jnp.int32, L)

        def pad_body(i, _):
            off = (i * L) % (H - N - L)
            idx_v[pl.ds(TAIL + i * L, L)] = PAD_IDX + off + lane
            return 0

        lax.fori_loop(0, (CHUNK - TAIL) // L, pad_body, 0, unroll=8)

    plsc.subcore_barrier()

    pltpu.sync_copy(ones_v, hist_sh.at[idx_v], add=True)
    plsc.subcore_barrier()

    @pl.when(sid == 0)
    def _writeback():
        pltpu.sync_copy(hist_sh, out_hbm.at[cid])


def _tc_body(hist_ref, feat_ref, out_ref):
    g = pl.program_id(0)

    @pl.when(g == 0)
    def _init():
        out_ref[...] = jnp.zeros_like(out_ref)

    c = jnp.sum(hist_ref[...], axis=0, keepdims=True)
    f = feat_ref[...]

    @pl.when(g == G - 1)
    def _masked_tail():
        row = (G - 1) * C + lax.broadcasted_iota(jnp.int32, (C, D), 0)
        fm = jnp.where(row < N, f, 0.0)
        out_ref[...] += lax.dot_general(
            c, fm, (((1,), (0,)), ((), ())),
            preferred_element_type=jnp.float32)

    @pl.when(g != G - 1)
    def _plain():
        out_ref[...] += lax.dot_general(
            c, f, (((1,), (0,)), ((), ())),
            preferred_element_type=jnp.float32)

    @pl.when(g == G - 1)
    def _scale():
        out_ref[...] *= jnp.float32(1.0 / E)


_tc_matvec = pl.pallas_call(
    _tc_body,
    grid=(G,),
    in_specs=[
        pl.BlockSpec((NC, C), lambda g: (0, g)),
        pl.BlockSpec((C, D), lambda g: (g, 0)),
    ],
    out_specs=pl.BlockSpec((1, D), lambda g: (0, 0)),
    out_shape=jax.ShapeDtypeStruct((1, D), jnp.float32),
)


def kernel(u, neighs, features):
    hist = _sc_hist(neighs.astype(jnp.int32))
    out = _tc_matvec(hist, features)
    return out[0]


# speedup vs baseline: 1.0934x; 1.0070x over previous
import functools

import jax
import jax.numpy as jnp
from jax import lax
from jax.experimental import pallas as pl
from jax.experimental.pallas import tpu as pltpu
from jax.experimental.pallas import tpu_sc as plsc

E = 500000
N = 100000
D = 128
NC, NS, L = 2, 16, 16
NW = NC * NS
CHUNK = 15744
TAIL = E - (NW - 1) * CHUNK
PAD_IDX = N
H = 102400
HS = H // NS
C = 12800
G = H // C


_sc_mesh = plsc.VectorSubcoreMesh(
    core_axis_name="c", subcore_axis_name="s", num_cores=NC, num_subcores=NS)


@functools.partial(
    pl.kernel,
    out_type=jax.ShapeDtypeStruct((NC, H), jnp.float32),
    mesh=_sc_mesh,
    scratch_types=[
        pltpu.VMEM((CHUNK,), jnp.int32),
        pltpu.VMEM((CHUNK,), jnp.float32),
        pltpu.VMEM((HS,), jnp.float32),
        pltpu.VMEM_SHARED((H,), jnp.float32),
        pltpu.SemaphoreType.DMA,
    ],
    compiler_params=pltpu.CompilerParams(needs_layout_passes=False),
)
def _sc_hist(neighs_hbm, out_hbm, idx_v, ones_v, zbuf_v, hist_sh, sem):
    cid = lax.axis_index("c")
    sid = lax.axis_index("s")
    wid = sid * NC + cid

    @pl.when(wid < NW - 1)
    def _load_full():
        pltpu.async_copy(neighs_hbm.at[pl.ds(wid * CHUNK, CHUNK)], idx_v, sem)

    @pl.when(wid == NW - 1)
    def _load_tail():
        pltpu.async_copy(neighs_hbm.at[pl.ds((NW - 1) * CHUNK, TAIL)],
                         idx_v.at[pl.ds(0, TAIL)], sem)

    ones = jnp.full((L,), 1.0, jnp.float32)
    zeros = jnp.zeros((L,), jnp.float32)

    def fill_body(j, _):
        ones_v[pl.ds(j * L, L)] = ones
        return 0

    lax.fori_loop(0, CHUNK // L, fill_body, 0, unroll=8)

    def zero_body(i, _):
        zbuf_v[pl.ds(i * L, L)] = zeros
        return 0

    lax.fori_loop(0, HS // L, zero_body, 0, unroll=8)
    pltpu.sync_copy(zbuf_v, hist_sh.at[pl.ds(sid * HS, HS)])

    @pl.when(wid < NW - 1)
    def _wait_full():
        pltpu.make_async_copy(
            neighs_hbm.at[pl.ds(wid * CHUNK, CHUNK)], idx_v, sem).wait()

    @pl.when(wid == NW - 1)
    def _wait_tail():
        pltpu.make_async_copy(
            neighs_hbm.at[pl.ds((NW - 1) * CHUNK, TAIL)],
            idx_v.at[pl.ds(0, TAIL)], sem).wait()
        lane = lax.iota(jnp.int32, L)

        def pad_body(i, _):
            off = (i * L) % (H - N - L)
            idx_v[pl.ds(TAIL + i * L, L)] = PAD_IDX + off + lane
            return 0

        lax.fori_loop(0, (CHUNK - TAIL) // L, pad_body, 0, unroll=8)

    plsc.subcore_barrier()

    pltpu.sync_copy(ones_v, hist_sh.at[idx_v], add=True)
    plsc.subcore_barrier()

    @pl.when(sid == 0)
    def _writeback():
        pltpu.sync_copy(hist_sh, out_hbm.at[cid])


def _tc_body(hist_ref, feat_ref, out_ref):
    g = pl.program_id(0)

    @pl.when(g == 0)
    def _init():
        out_ref[...] = jnp.zeros_like(out_ref)

    c = jnp.sum(hist_ref[...], axis=0, keepdims=True)
    f = feat_ref[...]
    row = g * C + lax.broadcasted_iota(jnp.int32, (C, D), 0)
    f = jnp.where(row < N, f, 0.0)
    out_ref[...] += lax.dot_general(
        c, f, (((1,), (0,)), ((), ())), preferred_element_type=jnp.float32)

    @pl.when(g == G - 1)
    def _scale():
        out_ref[...] *= jnp.float32(1.0 / E)


_tc_matvec = pl.pallas_call(
    _tc_body,
    grid=(G,),
    in_specs=[
        pl.BlockSpec((NC, C), lambda g: (0, g)),
        pl.BlockSpec((C, D), lambda g: (g, 0)),
    ],
    out_specs=pl.BlockSpec((1, D), lambda g: (0, 0)),
    out_shape=jax.ShapeDtypeStruct((1, D), jnp.float32),
)


def kernel(u, neighs, features):
    hist = _sc_hist(neighs.astype(jnp.int32))
    out = _tc_matvec(hist, features)
    return out[0]
